# baseline (speedup 1.0000x reference)
import jax
import jax.numpy as jnp
from jax import lax
from jax.experimental import pallas as pl
from jax.experimental.pallas import tpu as pltpu
from jax.experimental.pallas import tpu_sc as plsc

_K = 32
_EPS = 1e-7
_B, _N = 64, 8192
_L = 16
_NV = _N // _L
_NS = 8
_VPS = _NV // _NS
_ROWS_PER_W = 2
_UNROLL = 4


def _merge32(a0, a1, b0, b1):
    l0 = jnp.maximum(a0, lax.rev(b1, (0,)))
    l1 = jnp.maximum(a1, lax.rev(b0, (0,)))
    w = jnp.minimum(l0, l1)
    u = jnp.maximum(l0, l1)
    return w, u


def _row_compute(row_v, out_v):
    ninf = jnp.full((_L,), -jnp.inf, jnp.float32)

    def mstep(i, carry):
        new = []
        for s in range(_NS):
            a0, a1 = carry[2 * s], carry[2 * s + 1]
            v = row_v[pl.ds((s * _VPS + i) * _L, _L)]
            rs = lax.rev(lax.sort(v), (0,))
            l0 = jnp.maximum(a0, rs)
            w = jnp.minimum(l0, a1)
            u = jnp.maximum(l0, a1)
            new.append(lax.sort(w))
            new.append(lax.sort(u))
        return tuple(new)
    carry = lax.fori_loop(0, _VPS, mstep, (ninf,) * (2 * _NS))

    bufs = [(carry[2 * s], carry[2 * s + 1]) for s in range(_NS)]
    while len(bufs) > 2:
        nxt = []
        for p in range(0, len(bufs), 2):
            (a0, a1), (b0, b1) = bufs[p], bufs[p + 1]
            w, u = _merge32(a0, a1, b0, b1)
            nxt.append((lax.sort(w), lax.sort(u)))
        bufs = nxt
    (a0, a1), (b0, b1) = bufs
    w, u = _merge32(a0, a1, b0, b1)

    kth = jnp.min(w)
    delta = jnp.full((_L,), kth, jnp.float32) + _EPS

    acc = jnp.maximum(w - delta, 0.0) + jnp.maximum(u - delta, 0.0)
    s_vec = jnp.full((_L,), jnp.sum(acc), jnp.float32) + _EPS
    inv = jnp.ones((_L,), jnp.float32) / s_vec

    def fstep(i, c):
        for j in range(_UNROLL):
            o = (i * _UNROLL + j) * _L
            v = row_v[pl.ds(o, _L)]
            out_v[pl.ds(o, _L)] = jnp.maximum(v - delta, 0.0) * inv
        return c
    lax.fori_loop(0, _NV // _UNROLL, fstep, 0)


def _sc_body(x_hbm, out_hbm, row0_v, row1_v, out0_v, out1_v,
             sem0, sem1, osem0, osem1):
    wid = lax.axis_index("s") * 2 + lax.axis_index("c")
    r0 = wid * _ROWS_PER_W
    cp0 = pltpu.async_copy(x_hbm.at[r0], row0_v, sem0)
    cp1 = pltpu.async_copy(x_hbm.at[r0 + 1], row1_v, sem1)

    cp0.wait()
    _row_compute(row0_v, out0_v)
    ocp0 = pltpu.async_copy(out0_v, out_hbm.at[r0], osem0)

    cp1.wait()
    _row_compute(row1_v, out1_v)
    ocp1 = pltpu.async_copy(out1_v, out_hbm.at[r0 + 1], osem1)

    ocp0.wait()
    ocp1.wait()


@jax.jit
def _sc_call(attn_s):
    mesh = plsc.VectorSubcoreMesh(core_axis_name="c", subcore_axis_name="s")
    return pl.kernel(
        _sc_body,
        out_type=jax.ShapeDtypeStruct((_B, _N), jnp.float32),
        mesh=mesh,
        compiler_params=pltpu.CompilerParams(needs_layout_passes=False),
        scratch_types=[
            pltpu.VMEM((_N,), jnp.float32),
            pltpu.VMEM((_N,), jnp.float32),
            pltpu.VMEM((_N,), jnp.float32),
            pltpu.VMEM((_N,), jnp.float32),
            pltpu.SemaphoreType.DMA,
            pltpu.SemaphoreType.DMA,
            pltpu.SemaphoreType.DMA,
            pltpu.SemaphoreType.DMA,
        ],
    )(attn_s)


def kernel(attn_s):
    return _sc_call(attn_s)
